# Initial kernel scaffold; baseline (speedup 1.0000x reference)
import jax
import jax.numpy as jnp
from jax.experimental import pallas as pl


def kernel(news_x, edge_index, edge_attr, num_companies):
    raise NotImplementedError("write your pallas kernel here")


# speedup vs baseline: 3.6345x; 3.6345x over previous
import jax
import jax.numpy as jnp
from jax import lax
from jax.experimental import pallas as pl
from jax.experimental.pallas import tpu as pltpu
from jax.experimental.pallas import tpu_sc as plsc
import functools

N_NODES = 10000
N_PAD = 10240
D_FEAT = 256
DH = 128
N_EDGES = 160000
NC = 2
NS = 16
EPT = N_EDGES // NS
B = 40
NCH = EPT // B
RPT = N_PAD // NS
RB = 32
NRB = RPT // RB
N_TAIL = N_NODES % RB


def _body(news_ref, src_ref, dst_ref, attr_ref, out_ref,
          acc, ws, src_v, dst_v, attr_v, rows_v, blk_v, ws_v, sem):
    c = lax.axis_index("c")
    s = lax.axis_index("s")
    zero16 = jnp.zeros((16,), jnp.float32)

    def zrow(i, _):
        for v in range(DH // 16):
            blk_v[i, pl.ds(v * 16, 16)] = zero16
        return 0
    lax.fori_loop(0, RB, zrow, 0)
    r0t = s * RPT
    for k in range(NRB):
        pltpu.sync_copy(blk_v, acc.at[pl.ds(r0t + k * RB, RB)])
    for k in range(RPT // DH):
        pltpu.sync_copy(blk_v.at[0], ws.at[pl.ds(r0t + k * DH, DH)])

    e0 = s * EPT
    pltpu.sync_copy(src_ref.at[pl.ds(e0, EPT)], src_v)
    pltpu.sync_copy(attr_ref.at[pl.ds(e0, EPT)], attr_v.at[pl.ds(0, EPT)])

    def mkidx(i, _):
        sl = pl.ds(i * 16, 16)
        src_v[sl] = src_v[sl] * 2 + c
        return 0
    lax.fori_loop(0, EPT // 16, mkidx, 0)

    plsc.subcore_barrier()

    def chunk(j, _):
        off = j * B
        idx = src_v.at[pl.ds(off, B)]
        pltpu.sync_copy(dst_ref.at[s, j], dst_v)
        pltpu.async_copy(news_ref.at[idx], rows_v, sem).wait()

        def scale(e, _):
            a16 = attr_v[pl.ds(off + e, 16)]
            av = jnp.broadcast_to(a16[0], (16,))
            for v in range(DH // 16):
                sl = pl.ds(v * 16, 16)
                rows_v[e, sl] = rows_v[e, sl] * av
            return 0
        lax.fori_loop(0, B, scale, 0)

        pltpu.sync_copy(rows_v, acc.at[dst_v], add=True)
        pltpu.sync_copy(attr_v.at[pl.ds(off, B)], ws.at[dst_v], add=True)
        return 0
    lax.fori_loop(0, NCH, chunk, 0)

    plsc.subcore_barrier()

    for k in range(NRB):
        r0 = r0t + k * RB
        pltpu.sync_copy(acc.at[pl.ds(r0, RB)], blk_v)
        pltpu.sync_copy(ws.at[pl.ds(r0, RB)], ws_v.at[pl.ds(0, RB)])

        def fin(r, _):
            w16 = ws_v[pl.ds(r, 16)]
            w = jnp.broadcast_to(w16[0], (16,))
            iv = 1.0 / jnp.maximum(w, 1e-9)
            for v in range(DH // 16):
                sl = pl.ds(v * 16, 16)
                blk_v[r, sl] = blk_v[r, sl] * iv
            return 0
        lax.fori_loop(0, RB, fin, 0)

        @pl.when(r0 + RB <= N_NODES)
        def _():
            pltpu.sync_copy(blk_v, out_ref.at[pl.ds(r0, RB), pl.ds(c * DH, DH)])

        @pl.when(jnp.logical_and(r0 < N_NODES, r0 + RB > N_NODES))
        def _():
            pltpu.sync_copy(blk_v.at[pl.ds(0, N_TAIL)],
                            out_ref.at[pl.ds(r0, N_TAIL), pl.ds(c * DH, DH)])


@functools.partial(jax.jit, static_argnums=())
def _run(news_flat, src, dst3, attr):
    mesh = plsc.VectorSubcoreMesh(core_axis_name="c", subcore_axis_name="s",
                                  num_cores=NC, num_subcores=NS)
    f = pl.kernel(
        _body,
        out_type=jax.ShapeDtypeStruct((N_NODES, D_FEAT), jnp.float32),
        mesh=mesh,
        scratch_types=[
            pltpu.VMEM_SHARED((N_PAD, DH), jnp.float32),
            pltpu.VMEM_SHARED((N_PAD,), jnp.float32),
            pltpu.VMEM((EPT,), jnp.int32),
            pltpu.VMEM((B,), jnp.int32),
            pltpu.VMEM((EPT + 16,), jnp.float32),
            pltpu.VMEM((B, DH), jnp.float32),
            pltpu.VMEM((RB, DH), jnp.float32),
            pltpu.VMEM((RB + 16,), jnp.float32),
            pltpu.SemaphoreType.DMA,
        ],
    )
    return f(news_flat, src, dst3, attr)


def kernel(news_x, edge_index, edge_attr, num_companies):
    del num_companies
    news_flat = news_x.reshape(2 * N_NODES, DH)
    src = edge_index[0].astype(jnp.int32)
    dst3 = edge_index[1].astype(jnp.int32).reshape(NS, NCH, B)
    return _run(news_flat, src, dst3, edge_attr)


# speedup vs baseline: 7.9471x; 2.1866x over previous
import jax
import jax.numpy as jnp
from jax import lax
from jax.experimental import pallas as pl
from jax.experimental.pallas import tpu as pltpu
from jax.experimental.pallas import tpu_sc as plsc
import functools

N_NODES = 10000
N_PAD = 10240
D_FEAT = 256
DH = 128
N_EDGES = 160000
NC = 2
NS = 16
EPT = N_EDGES // NS
B = 40
NCH = EPT // B
RPT = N_PAD // NS
RB = 32
NRB = RPT // RB
N_TAIL = N_NODES % RB


def _body(news_ref, src_ref, dst_ref, attr_ref, out_ref,
          acc, ws, src_v, dst0, dst1, attr_v, rows0, rows1, blk_v, ws_v,
          gsem0, gsem1, ssem0, ssem1):
    c = lax.axis_index("c")
    s = lax.axis_index("s")
    zero16 = jnp.zeros((16,), jnp.float32)

    def zrow(i, _):
        for v in range(DH // 16):
            blk_v[i, pl.ds(v * 16, 16)] = zero16
        return 0
    lax.fori_loop(0, RB, zrow, 0)
    r0t = s * RPT
    for k in range(NRB):
        pltpu.sync_copy(blk_v, acc.at[pl.ds(r0t + k * RB, RB)])
    for k in range(RPT // DH):
        pltpu.sync_copy(blk_v.at[0], ws.at[pl.ds(r0t + k * DH, DH)])

    e0 = s * EPT
    pltpu.sync_copy(src_ref.at[pl.ds(e0, EPT)], src_v)
    pltpu.sync_copy(attr_ref.at[pl.ds(e0, EPT)], attr_v.at[pl.ds(0, EPT)])

    def mkidx(i, _):
        sl = pl.ds(i * 16, 16)
        src_v[sl] = src_v[sl] * 2 + c
        return 0
    lax.fori_loop(0, EPT // 16, mkidx, 0)

    plsc.subcore_barrier()

    bufs = ((rows0, dst0, gsem0, ssem0), (rows1, dst1, gsem1, ssem1))

    pltpu.async_copy(news_ref.at[src_v.at[pl.ds(0, B)]], rows0, gsem0)
    pltpu.async_copy(dst_ref.at[s, 0], dst0, gsem0)

    def outer(j2, _):
        for p in range(2):
            rows_p, dst_p, gsem_p, ssem_p = bufs[p]
            rows_q, dst_q, gsem_q, ssem_q = bufs[1 - p]
            j = j2 * 2 + p
            off = j * B

            @pl.when(j > 0)
            def _():
                pltpu.make_async_copy(rows_q, acc.at[dst_q], ssem_q).wait()
                pltpu.make_async_copy(attr_v.at[pl.ds(0, B)],
                                      ws.at[dst_q], ssem_q).wait()

            @pl.when(j + 1 < NCH)
            def _():
                pltpu.async_copy(news_ref.at[src_v.at[pl.ds(off + B, B)]],
                                 rows_q, gsem_q)
                pltpu.async_copy(dst_ref.at[s, j + 1], dst_q, gsem_q)

            pltpu.make_async_copy(news_ref.at[src_v.at[pl.ds(off, B)]],
                                  rows_p, gsem_p).wait()
            pltpu.make_async_copy(dst_ref.at[s, j], dst_p, gsem_p).wait()

            def scale(e, _):
                a16 = attr_v[pl.ds(off + e, 16)]
                av = jnp.broadcast_to(a16[0], (16,))
                for v in range(DH // 16):
                    sl = pl.ds(v * 16, 16)
                    rows_p[e, sl] = rows_p[e, sl] * av
                return 0
            lax.fori_loop(0, B, scale, 0)

            pltpu.async_copy(rows_p, acc.at[dst_p], ssem_p, add=True)
            pltpu.async_copy(attr_v.at[pl.ds(off, B)], ws.at[dst_p],
                             ssem_p, add=True)
        return 0
    lax.fori_loop(0, NCH // 2, outer, 0)

    pltpu.make_async_copy(rows1, acc.at[dst1], ssem1).wait()
    pltpu.make_async_copy(attr_v.at[pl.ds(0, B)], ws.at[dst1], ssem1).wait()

    plsc.subcore_barrier()

    for k in range(NRB):
        r0 = r0t + k * RB
        pltpu.sync_copy(acc.at[pl.ds(r0, RB)], blk_v)
        pltpu.sync_copy(ws.at[pl.ds(r0, RB)], ws_v.at[pl.ds(0, RB)])

        def fin(r, _):
            w16 = ws_v[pl.ds(r, 16)]
            w = jnp.broadcast_to(w16[0], (16,))
            iv = 1.0 / jnp.maximum(w, 1e-9)
            for v in range(DH // 16):
                sl = pl.ds(v * 16, 16)
                blk_v[r, sl] = blk_v[r, sl] * iv
            return 0
        lax.fori_loop(0, RB, fin, 0)

        @pl.when(r0 + RB <= N_NODES)
        def _():
            pltpu.sync_copy(blk_v, out_ref.at[pl.ds(r0, RB), pl.ds(c * DH, DH)])

        @pl.when(jnp.logical_and(r0 < N_NODES, r0 + RB > N_NODES))
        def _():
            pltpu.sync_copy(blk_v.at[pl.ds(0, N_TAIL)],
                            out_ref.at[pl.ds(r0, N_TAIL), pl.ds(c * DH, DH)])


@functools.partial(jax.jit, static_argnums=())
def _run(news_flat, src, dst3, attr):
    mesh = plsc.VectorSubcoreMesh(core_axis_name="c", subcore_axis_name="s",
                                  num_cores=NC, num_subcores=NS)
    f = pl.kernel(
        _body,
        out_type=jax.ShapeDtypeStruct((N_NODES, D_FEAT), jnp.float32),
        mesh=mesh,
        scratch_types=[
            pltpu.VMEM_SHARED((N_PAD, DH), jnp.float32),
            pltpu.VMEM_SHARED((N_PAD,), jnp.float32),
            pltpu.VMEM((EPT,), jnp.int32),
            pltpu.VMEM((B,), jnp.int32),
            pltpu.VMEM((B,), jnp.int32),
            pltpu.VMEM((EPT + 16,), jnp.float32),
            pltpu.VMEM((B, DH), jnp.float32),
            pltpu.VMEM((B, DH), jnp.float32),
            pltpu.VMEM((RB, DH), jnp.float32),
            pltpu.VMEM((RB + 16,), jnp.float32),
            pltpu.SemaphoreType.DMA,
            pltpu.SemaphoreType.DMA,
            pltpu.SemaphoreType.DMA,
            pltpu.SemaphoreType.DMA,
        ],
    )
    return f(news_flat, src, dst3, attr)


def kernel(news_x, edge_index, edge_attr, num_companies):
    del num_companies
    news_flat = news_x.reshape(2 * N_NODES, DH)
    src = edge_index[0].astype(jnp.int32)
    dst3 = edge_index[1].astype(jnp.int32).reshape(NS, NCH, B)
    return _run(news_flat, src, dst3, edge_attr)


# speedup vs baseline: 11.3149x; 1.4238x over previous
import jax
import jax.numpy as jnp
from jax import lax
from jax.experimental import pallas as pl
from jax.experimental.pallas import tpu as pltpu
from jax.experimental.pallas import tpu_sc as plsc
import functools

N_NODES = 10000
N_PAD = 10240
D_FEAT = 256
DH = 128
N_EDGES = 160000
NC = 2
NS = 16
EPT = N_EDGES // NS
B = 40
NCH = EPT // B
NBUF = 4
NCH_MAIN = NCH - 2
RPT = N_PAD // NS
RB = 32
NRB = RPT // RB
N_TAIL = N_NODES % RB


def _body(news_ref, src_ref, dst_ref, attr_ref, out_ref,
          acc, ws, src_v,
          rows0, rows1, rows2, rows3,
          dst0, dst1, dst2, dst3,
          ab0, ab1, ab2, ab3,
          blk_v, ws_v,
          sem0, sem1, sem2, sem3):
    c = lax.axis_index("c")
    s = lax.axis_index("s")
    zero16 = jnp.zeros((16,), jnp.float32)
    rows = (rows0, rows1, rows2, rows3)
    dstb = (dst0, dst1, dst2, dst3)
    attrb = (ab0, ab1, ab2, ab3)
    sems = (sem0, sem1, sem2, sem3)
    e0 = s * EPT

    def fetch(j, p):
        pltpu.async_copy(news_ref.at[src_v.at[pl.ds(j * B, B)]],
                         rows[p], sems[p])
        pltpu.async_copy(dst_ref.at[s, j], dstb[p], sems[p])
        pltpu.async_copy(attr_ref.at[pl.ds(e0 + j * B, B)],
                         attrb[p].at[pl.ds(0, B)], sems[p])

    def wait_fetch(j, p):
        pltpu.make_async_copy(news_ref.at[src_v.at[pl.ds(j * B, B)]],
                              rows[p], sems[p]).wait()
        pltpu.make_async_copy(dst_ref.at[s, j], dstb[p], sems[p]).wait()
        pltpu.make_async_copy(attr_ref.at[pl.ds(e0 + j * B, B)],
                              attrb[p].at[pl.ds(0, B)], sems[p]).wait()

    def scatter(p):
        pltpu.async_copy(rows[p], acc.at[dstb[p]], sems[p], add=True)
        pltpu.async_copy(attrb[p].at[pl.ds(0, B)], ws.at[dstb[p]],
                         sems[p], add=True)

    def wait_scatter(p):
        pltpu.make_async_copy(rows[p], acc.at[dstb[p]], sems[p]).wait()
        pltpu.make_async_copy(attrb[p].at[pl.ds(0, B)], ws.at[dstb[p]],
                              sems[p]).wait()

    def scale(p):
        rows_p, attr_p = rows[p], attrb[p]

        def body(e, _):
            a16 = attr_p[pl.ds(e, 16)]
            av = jnp.broadcast_to(a16[0], (16,))
            for v in range(DH // 16):
                sl = pl.ds(v * 16, 16)
                rows_p[e, sl] = rows_p[e, sl] * av
            return 0
        lax.fori_loop(0, B, body, 0)

    def zrow(i, _):
        for v in range(DH // 16):
            blk_v[i, pl.ds(v * 16, 16)] = zero16
        return 0
    lax.fori_loop(0, RB, zrow, 0)
    r0t = s * RPT
    for k in range(NRB):
        pltpu.sync_copy(blk_v, acc.at[pl.ds(r0t + k * RB, RB)])
    for k in range(RPT // DH):
        pltpu.sync_copy(blk_v.at[0], ws.at[pl.ds(r0t + k * DH, DH)])

    pltpu.sync_copy(src_ref.at[pl.ds(e0, EPT)], src_v)

    def mkidx(i, _):
        sl = pl.ds(i * 16, 16)
        src_v[sl] = src_v[sl] * 2 + c
        return 0
    lax.fori_loop(0, EPT // 16, mkidx, 0)

    plsc.subcore_barrier()

    fetch(0, 0)
    fetch(1, 1)

    def outer(j4, _):
        for p in range(NBUF):
            j = j4 * NBUF + p
            pn = (p + 2) % NBUF

            @pl.when(j >= 2)
            def _():
                wait_scatter(pn)
            fetch(j + 2, pn)
            wait_fetch(j, p)
            scale(p)
            scatter(p)
        return 0
    lax.fori_loop(0, NCH_MAIN // NBUF, outer, 0)

    for jt in range(NCH_MAIN, NCH):
        p = jt % NBUF
        wait_scatter((p + 2) % NBUF)
        wait_fetch(jt, p)
        scale(p)
        scatter(p)
    for jt in range(NCH_MAIN, NCH):
        wait_scatter(jt % NBUF)

    plsc.subcore_barrier()

    for k in range(NRB):
        r0 = r0t + k * RB
        pltpu.sync_copy(acc.at[pl.ds(r0, RB)], blk_v)
        pltpu.sync_copy(ws.at[pl.ds(r0, RB)], ws_v.at[pl.ds(0, RB)])

        def fin(r, _):
            w16 = ws_v[pl.ds(r, 16)]
            w = jnp.broadcast_to(w16[0], (16,))
            iv = 1.0 / jnp.maximum(w, 1e-9)
            for v in range(DH // 16):
                sl = pl.ds(v * 16, 16)
                blk_v[r, sl] = blk_v[r, sl] * iv
            return 0
        lax.fori_loop(0, RB, fin, 0)

        @pl.when(r0 + RB <= N_NODES)
        def _():
            pltpu.sync_copy(blk_v, out_ref.at[pl.ds(r0, RB), pl.ds(c * DH, DH)])

        @pl.when(jnp.logical_and(r0 < N_NODES, r0 + RB > N_NODES))
        def _():
            pltpu.sync_copy(blk_v.at[pl.ds(0, N_TAIL)],
                            out_ref.at[pl.ds(r0, N_TAIL), pl.ds(c * DH, DH)])


@functools.partial(jax.jit, static_argnums=())
def _run(news_flat, src, dst3, attr):
    mesh = plsc.VectorSubcoreMesh(core_axis_name="c", subcore_axis_name="s",
                                  num_cores=NC, num_subcores=NS)
    f = pl.kernel(
        _body,
        out_type=jax.ShapeDtypeStruct((N_NODES, D_FEAT), jnp.float32),
        mesh=mesh,
        scratch_types=[
            pltpu.VMEM_SHARED((N_PAD, DH), jnp.float32),
            pltpu.VMEM_SHARED((N_PAD,), jnp.float32),
            pltpu.VMEM((EPT,), jnp.int32),
            pltpu.VMEM((B, DH), jnp.float32),
            pltpu.VMEM((B, DH), jnp.float32),
            pltpu.VMEM((B, DH), jnp.float32),
            pltpu.VMEM((B, DH), jnp.float32),
            pltpu.VMEM((B,), jnp.int32),
            pltpu.VMEM((B,), jnp.int32),
            pltpu.VMEM((B,), jnp.int32),
            pltpu.VMEM((B,), jnp.int32),
            pltpu.VMEM((B + 16,), jnp.float32),
            pltpu.VMEM((B + 16,), jnp.float32),
            pltpu.VMEM((B + 16,), jnp.float32),
            pltpu.VMEM((B + 16,), jnp.float32),
            pltpu.VMEM((RB, DH), jnp.float32),
            pltpu.VMEM((RB + 16,), jnp.float32),
            pltpu.SemaphoreType.DMA,
            pltpu.SemaphoreType.DMA,
            pltpu.SemaphoreType.DMA,
            pltpu.SemaphoreType.DMA,
        ],
    )
    return f(news_flat, src, dst3, attr)


def kernel(news_x, edge_index, edge_attr, num_companies):
    del num_companies
    news_flat = news_x.reshape(2 * N_NODES, DH)
    src = edge_index[0].astype(jnp.int32)
    dst3 = edge_index[1].astype(jnp.int32).reshape(NS, NCH, B)
    return _run(news_flat, src, dst3, edge_attr)
